# baseline (speedup 1.0000x reference)
import functools

import jax
import jax.numpy as jnp
from jax import lax
from jax.experimental import pallas as pl
from jax.experimental.pallas import tpu as pltpu
from jax.experimental.pallas import tpu_sc as plsc

_NUM_CORES = 2
_NUM_SUBCORES = 16
_NW = _NUM_CORES * _NUM_SUBCORES
_CHUNK = 640
_NBUF = 2


@functools.lru_cache(maxsize=None)
def _build_gather(n, v, d):
    per_w = n // _NW
    nchunk = per_w // _CHUNK
    assert per_w % _CHUNK == 0 and nchunk % _NBUF == 0

    mesh = plsc.VectorSubcoreMesh(core_axis_name="c", subcore_axis_name="s")

    @functools.partial(
        pl.kernel,
        mesh=mesh,
        out_type=jax.ShapeDtypeStruct((n, d), jnp.float32),
        compiler_params=pltpu.CompilerParams(use_tc_tiling_on_sc=False),
        scratch_types=[
            pltpu.VMEM((per_w,), jnp.int32),
            pltpu.VMEM((_NBUF, _CHUNK, d), jnp.float32),
            [pltpu.SemaphoreType.DMA] * _NBUF,
            [pltpu.SemaphoreType.DMA] * _NBUF,
        ],
    )
    def gather_kernel(table_hbm, idx_hbm, out_hbm, idx_v, rows_v, gsems, wsems):
        wid = lax.axis_index("s") * _NUM_CORES + lax.axis_index("c")
        base = wid * per_w
        pltpu.sync_copy(idx_hbm.at[wid], idx_v)

        def chunk_ops(j, b):
            gather = pltpu.make_async_copy(
                table_hbm.at[idx_v.at[pl.ds(j * _CHUNK, _CHUNK)]],
                rows_v.at[b],
                gsems[b],
            )
            write = pltpu.make_async_copy(
                rows_v.at[b],
                out_hbm.at[pl.ds(base + j * _CHUNK, _CHUNK)],
                wsems[b],
            )
            return gather, write

        def body(go, carry):
            j0 = go * _NBUF
            for b in range(_NBUF):
                g, w = chunk_ops(j0 + b, b)

                @pl.when(go > 0)
                def _():
                    w.wait()

                g.start()
            for b in range(_NBUF):
                g, w = chunk_ops(j0 + b, b)
                g.wait()
                w.start()
            return carry

        lax.fori_loop(0, nchunk // _NBUF, body, 0)

        for b in range(_NBUF):
            _, w = chunk_ops(nchunk - _NBUF + b, b)
            w.wait()

    return gather_kernel


def kernel(xs, weights):
    b, h = xs.shape
    v, d = weights.shape
    n = b * h
    idx = xs.reshape(_NW, n // _NW).astype(jnp.int32)
    idx, weights = lax.optimization_barrier((idx, weights))
    out = _build_gather(n, v, d)(weights, idx)
    out = lax.optimization_barrier(out)
    return out.reshape(b, h, d)


# speedup vs baseline: 1.4645x; 1.4645x over previous
import functools

import jax
import jax.numpy as jnp
from jax import lax
from jax.experimental import pallas as pl
from jax.experimental.pallas import tpu as pltpu
from jax.experimental.pallas import tpu_sc as plsc

_NUM_CORES = 2
_NUM_SUBCORES = 16
_NW = _NUM_CORES * _NUM_SUBCORES
_CHUNK = 640
_NBUF = 2


@functools.lru_cache(maxsize=None)
def _build_gather(n, v, d):
    per_w = n // _NW
    nchunk = per_w // _CHUNK
    assert per_w % _CHUNK == 0 and nchunk % _NBUF == 0

    mesh = plsc.VectorSubcoreMesh(core_axis_name="c", subcore_axis_name="s")

    @functools.partial(
        pl.kernel,
        mesh=mesh,
        out_type=jax.ShapeDtypeStruct((n, d), jnp.float32),
        compiler_params=pltpu.CompilerParams(use_tc_tiling_on_sc=False),
        scratch_types=[
            pltpu.VMEM((per_w,), jnp.int32),
            pltpu.VMEM((_NBUF, _CHUNK, d), jnp.float32),
            [pltpu.SemaphoreType.DMA] * _NBUF,
            [pltpu.SemaphoreType.DMA] * _NBUF,
        ],
    )
    def gather_kernel(table_hbm, idx_hbm, out_hbm, idx_v, rows_v, gsems, wsems):
        wid = lax.axis_index("s") * _NUM_CORES + lax.axis_index("c")
        base = wid * per_w
        pltpu.sync_copy(idx_hbm.at[wid], idx_v)

        def chunk_ops(j, b):
            gather = pltpu.make_async_copy(
                table_hbm.at[idx_v.at[pl.ds(j * _CHUNK, _CHUNK)]],
                rows_v.at[b],
                gsems[b],
            )
            write = pltpu.make_async_copy(
                rows_v.at[b],
                out_hbm.at[pl.ds(base + j * _CHUNK, _CHUNK)],
                wsems[b],
            )
            return gather, write

        def body(go, carry):
            j0 = go * _NBUF
            for b in range(_NBUF):
                g, w = chunk_ops(j0 + b, b)

                @pl.when(go > 0)
                def _():
                    w.wait()

                g.start()
            for b in range(_NBUF):
                g, w = chunk_ops(j0 + b, b)
                g.wait()
                w.start()
            return carry

        lax.fori_loop(0, nchunk // _NBUF, body, 0)

        for b in range(_NBUF):
            _, w = chunk_ops(nchunk - _NBUF + b, b)
            w.wait()

    return gather_kernel


def kernel(xs, weights):
    b, h = xs.shape
    v, d = weights.shape
    n = b * h
    idx = xs.reshape(_NW, n // _NW).astype(jnp.int32)
    idx = lax.optimization_barrier(idx)
    out = _build_gather(n, v, d)(weights, idx)
    return out.reshape(b, h, d)


# speedup vs baseline: 1.4677x; 1.0022x over previous
import functools

import jax
import jax.numpy as jnp
from jax import lax
from jax.experimental import pallas as pl
from jax.experimental.pallas import tpu as pltpu
from jax.experimental.pallas import tpu_sc as plsc

_NUM_CORES = 2
_NUM_SUBCORES = 16
_NW = _NUM_CORES * _NUM_SUBCORES
_CHUNK = 640
_NBUF = 2


@functools.lru_cache(maxsize=None)
def _build_gather(n, v, d):
    per_w = n // _NW
    nchunk = per_w // _CHUNK
    assert per_w % _CHUNK == 0 and nchunk % _NBUF == 0

    mesh = plsc.VectorSubcoreMesh(core_axis_name="c", subcore_axis_name="s")

    @functools.partial(
        pl.kernel,
        mesh=mesh,
        out_type=jax.ShapeDtypeStruct((n, d), jnp.float32),
        compiler_params=pltpu.CompilerParams(use_tc_tiling_on_sc=False),
        scratch_types=[
            pltpu.VMEM((per_w,), jnp.int32),
            pltpu.VMEM((_NBUF, _CHUNK, d), jnp.float32),
            [pltpu.SemaphoreType.DMA] * _NBUF,
            [pltpu.SemaphoreType.DMA] * _NBUF,
        ],
    )
    def gather_kernel(table_hbm, idx_hbm, out_hbm, idx_v, rows_v, gsems, wsems):
        wid = lax.axis_index("s") * _NUM_CORES + lax.axis_index("c")
        base = wid * per_w
        pltpu.sync_copy(idx_hbm.at[wid], idx_v)

        def chunk_ops(j, b):
            gather = pltpu.make_async_copy(
                table_hbm.at[idx_v.at[pl.ds(j * _CHUNK, _CHUNK)]],
                rows_v.at[b],
                gsems[b],
            )
            write = pltpu.make_async_copy(
                rows_v.at[b],
                out_hbm.at[pl.ds(base + j * _CHUNK, _CHUNK)],
                wsems[b],
            )
            return gather, write

        def body(go, carry):
            j0 = go * _NBUF
            for b in range(_NBUF):
                g, w = chunk_ops(j0 + b, b)

                @pl.when(go > 0)
                def _():
                    w.wait()

                g.start()
            for b in range(_NBUF):
                g, w = chunk_ops(j0 + b, b)
                g.wait()
                w.start()
            return carry

        lax.fori_loop(0, nchunk // _NBUF, body, 0)

        for b in range(_NBUF):
            _, w = chunk_ops(nchunk - _NBUF + b, b)
            w.wait()

    return gather_kernel


def kernel(xs, weights):
    b, h = xs.shape
    v, d = weights.shape
    n = b * h
    idx = xs.reshape(_NW, n // _NW).astype(jnp.int32)
    out = _build_gather(n, v, d)(weights, idx)
    return out.reshape(b, h, d)
